# Initial kernel scaffold; baseline (speedup 1.0000x reference)
import jax
import jax.numpy as jnp
from jax.experimental import pallas as pl


def kernel(x, edge_index, W1, b1, W2, b2):
    raise NotImplementedError("write your pallas kernel here")


# speedup vs baseline: 11.5794x; 11.5794x over previous
import functools

import jax
import jax.numpy as jnp
from jax import lax
from jax.experimental import pallas as pl
from jax.experimental.pallas import tpu as pltpu
from jax.experimental.pallas import tpu_sc as plsc

N = 10000
NPAD = 10240
E = 160000
D_IN = 256
D_HID = 512
NC, NS = 2, 16
NW = NC * NS
EPT = E // NW
B = 125
NB = EPT // B
CW = 128
NCHUNK = D_HID // CW
STRIPE = NPAD // NS

def _deg_body(col3, ones_h, zer_h, out2, colv, onesv, acc, sem):
    c = lax.axis_index("c")
    s = lax.axis_index("s")
    wid = s * NC + c
    pltpu.sync_copy(col3.at[wid], colv)
    pltpu.sync_copy(ones_h, onesv)
    pltpu.sync_copy(zer_h, acc.at[pl.ds(s * STRIPE, STRIPE)])
    plsc.subcore_barrier()

    def body(j, carry):
        pltpu.sync_copy(onesv, acc.at[colv.at[j]], add=True)
        return carry

    lax.fori_loop(0, NB, body, 0)
    plsc.subcore_barrier()
    pltpu.sync_copy(acc.at[pl.ds(s * STRIPE, STRIPE)],
                    out2.at[c, pl.ds(s * STRIPE, STRIPE)])


@functools.cache
def _deg_call():
    return pl.kernel(
        _deg_body,
        out_type=jax.ShapeDtypeStruct((NC, NPAD), jnp.float32),
        mesh=plsc.VectorSubcoreMesh(core_axis_name="c", subcore_axis_name="s"),
        scratch_types=[
            pltpu.VMEM((NB, B), jnp.int32),
            pltpu.VMEM((B,), jnp.float32),
            pltpu.VMEM_SHARED((NPAD,), jnp.float32),
            pltpu.SemaphoreType.DMA,
        ],
    )


def _scat_body(y4, row3, col3, zer2_h, out4,
               rowv, colv, buf0, buf1, acc, sem0, sem1):
    c = lax.axis_index("c")
    s = lax.axis_index("s")
    wid = s * NC + c
    pltpu.sync_copy(row3.at[wid], rowv)
    pltpu.sync_copy(col3.at[wid], colv)

    for k in range(NCHUNK):
        yk = y4.at[k]
        pltpu.sync_copy(zer2_h, acc.at[pl.ds(s * STRIPE, STRIPE)])
        plsc.subcore_barrier()

        pltpu.async_copy(yk.at[rowv.at[0]], buf0, sem0)

        def body(j2, carry):
            j = 2 * j2
            pltpu.make_async_copy(yk.at[rowv.at[j]], buf0, sem0).wait()
            pltpu.async_copy(yk.at[rowv.at[j + 1]], buf1, sem1)
            pltpu.sync_copy(buf0, acc.at[colv.at[j]], add=True)
            pltpu.make_async_copy(yk.at[rowv.at[j + 1]], buf1, sem1).wait()

            @pl.when(j2 + 1 < NB // 2)
            def _():
                pltpu.async_copy(yk.at[rowv.at[j + 2]], buf0, sem0)

            pltpu.sync_copy(buf1, acc.at[colv.at[j + 1]], add=True)
            return carry

        lax.fori_loop(0, NB // 2, body, 0)
        plsc.subcore_barrier()
        pltpu.sync_copy(acc.at[pl.ds(s * STRIPE, STRIPE)],
                        out4.at[c, k, pl.ds(s * STRIPE, STRIPE)])


@functools.cache
def _scat_call():
    return pl.kernel(
        _scat_body,
        out_type=jax.ShapeDtypeStruct((NC, NCHUNK, NPAD, CW), jnp.float32),
        mesh=plsc.VectorSubcoreMesh(core_axis_name="c", subcore_axis_name="s"),
        scratch_types=[
            pltpu.VMEM((NB, B), jnp.int32),
            pltpu.VMEM((NB, B), jnp.int32),
            pltpu.VMEM((B, CW), jnp.float32),
            pltpu.VMEM((B, CW), jnp.float32),
            pltpu.VMEM_SHARED((NPAD, CW), jnp.float32),
            pltpu.SemaphoreType.DMA,
            pltpu.SemaphoreType.DMA,
        ],
    )


_MT = 2000


def _d_from(degT_blk):
    deg = degT_blk[:, 0:1] + degT_blk[:, 1:2] + 1.0
    return lax.rsqrt(deg)


def _mm1_body(degT_ref, x_ref, w1_ref, out_ref):
    d = _d_from(degT_ref[...])
    xt = lax.dot_general(x_ref[...], w1_ref[...],
                         (((1,), (1,)), ((), ())),
                         preferred_element_type=jnp.float32)
    out_ref[0] = xt * d


def _mm1(degT, x, w1):
    return pl.pallas_call(
        _mm1_body,
        grid=(NCHUNK, N // _MT),
        in_specs=[
            pl.BlockSpec((_MT, 2), lambda n, m: (m, 0)),
            pl.BlockSpec((_MT, D_IN), lambda n, m: (m, 0)),
            pl.BlockSpec((CW, D_IN), lambda n, m: (n, 0)),
        ],
        out_specs=pl.BlockSpec((1, _MT, CW), lambda n, m: (n, m, 0)),
        out_shape=jax.ShapeDtypeStruct((NCHUNK, N, CW), jnp.float32),
    )(degT, x, w1)


def _mm2_body(degT_ref, s1_ref, y1_ref, b1_ref, w2t_ref, out_ref):
    d = _d_from(degT_ref[...])
    acc = jnp.zeros((_MT, CW), jnp.float32)
    for k in range(NCHUNK):
        hk = d * (s1_ref[0, k] + s1_ref[1, k] + y1_ref[k]) + b1_ref[k][None, :]
        hk = jnp.maximum(hk, 0.0)
        acc = acc + lax.dot_general(hk, w2t_ref[0, k * CW:(k + 1) * CW, :],
                                    (((1,), (0,)), ((), ())),
                                    preferred_element_type=jnp.float32)
    out_ref[0] = acc * d


def _mm2(degT, s1, y1, b1r, w2t):
    return pl.pallas_call(
        _mm2_body,
        grid=(NCHUNK, N // _MT),
        in_specs=[
            pl.BlockSpec((_MT, 2), lambda n, m: (m, 0)),
            pl.BlockSpec((NC, NCHUNK, _MT, CW), lambda n, m: (0, 0, m, 0)),
            pl.BlockSpec((NCHUNK, _MT, CW), lambda n, m: (0, m, 0)),
            pl.BlockSpec((NCHUNK, CW), lambda n, m: (0, 0)),
            pl.BlockSpec((1, D_HID, CW), lambda n, m: (n, 0, 0)),
        ],
        out_specs=pl.BlockSpec((1, _MT, CW), lambda n, m: (n, m, 0)),
        out_shape=jax.ShapeDtypeStruct((NCHUNK, N, CW), jnp.float32),
    )(degT, s1, y1, b1r, w2t)


def _ep3_body(degT_ref, s2_ref, y2_ref, b2_ref, out_ref):
    d = _d_from(degT_ref[...])
    b = b2_ref[pl.program_id(0)][None, :]
    out_ref[...] = d * (s2_ref[0, 0] + s2_ref[1, 0] + y2_ref[0]) + b


def _ep3(degT, s2, y2, b2r):
    return pl.pallas_call(
        _ep3_body,
        grid=(NCHUNK, N // _MT),
        in_specs=[
            pl.BlockSpec((_MT, 2), lambda n, m: (m, 0)),
            pl.BlockSpec((NC, 1, _MT, CW), lambda n, m: (0, n, m, 0)),
            pl.BlockSpec((1, _MT, CW), lambda n, m: (n, m, 0)),
            pl.BlockSpec((NCHUNK, CW), lambda n, m: (0, 0)),
        ],
        out_specs=pl.BlockSpec((_MT, CW), lambda n, m: (m, n)),
        out_shape=jax.ShapeDtypeStruct((N, D_HID), jnp.float32),
    )(degT, s2, y2, b2r)


@jax.jit
def kernel(x, edge_index, W1, b1, W2, b2):
    ei = edge_index.astype(jnp.int32)
    row3 = ei[0].reshape(NW, NB, B)
    col3 = ei[1].reshape(NW, NB, B)
    ones_h = jnp.ones((B,), jnp.float32)
    zer1 = jnp.zeros((STRIPE,), jnp.float32)
    zer2 = jnp.zeros((STRIPE, CW), jnp.float32)
    b1r = b1.reshape(NCHUNK, CW)
    b2r = b2.reshape(NCHUNK, CW)
    w2t = W2.T.reshape(D_HID, NCHUNK, CW).transpose(1, 0, 2)

    deg2 = _deg_call()(col3, ones_h, zer1)
    degT = deg2.T

    y1 = _mm1(degT, x, W1)
    s1 = _scat_call()(y1, row3, col3, zer2)
    y2 = _mm2(degT, s1, y1, b1r, w2t)
    s2 = _scat_call()(y2, row3, col3, zer2)
    return _ep3(degT, s2, y2, b2r)
